# Initial kernel scaffold; baseline (speedup 1.0000x reference)
import jax
import jax.numpy as jnp
from jax.experimental import pallas as pl


def kernel(support_embeddings, support_targets, query_embeddings, query_targets):
    raise NotImplementedError("write your pallas kernel here")


# speedup vs baseline: 4.0219x; 4.0219x over previous
import functools

import jax
import jax.numpy as jnp
from jax import lax
from jax.experimental import pallas as pl
from jax.experimental.pallas import tpu as pltpu

C = 64
B, N, Q, D = 16, 2048, 2048, 512
QT = 1024
NQ = Q // QT


def _tc_body(sup_t3_ref, q_ref, qt3_ref, sup_ref,
             dist_ref, pred_ref, loss_ref, acc_ref,
             protos_ref, p2_ref):
    b = pl.program_id(0)
    qi = pl.program_id(1)

    @pl.when(qi == 0)
    def _compute_protos():
        t = sup_t3_ref[0, 0, :]
        onehot = (lax.broadcasted_iota(jnp.int32, (C, N), 0)
                  == t[None, :]).astype(jnp.float32)
        counts = jnp.sum(onehot, axis=1, keepdims=True)
        sums = jnp.dot(onehot, sup_ref[0],
                       preferred_element_type=jnp.float32,
                       precision=lax.Precision.HIGHEST)
        protos = sums / jnp.maximum(counts, 1.0)
        protos_ref[...] = protos
        p2_ref[...] = jnp.sum(protos * protos, axis=1, keepdims=True)

    qblk = q_ref[0]
    q2 = jnp.sum(qblk * qblk, axis=1)
    protos = protos_ref[...]
    cross = lax.dot_general(protos, qblk, (((1,), (1,)), ((), ())),
                            preferred_element_type=jnp.float32)
    dist = p2_ref[...] + q2[None, :] - 2.0 * cross
    dist_ref[0] = dist

    logits = -dist
    mx = jnp.max(logits, axis=0, keepdims=True)
    se = jnp.sum(jnp.exp(logits - mx), axis=0, keepdims=True)
    lse = mx + jnp.log(se)
    tq = qt3_ref[0, 0, :]
    cls_iota = lax.broadcasted_iota(jnp.int32, (C, QT), 0)
    sel = jnp.sum(jnp.where(cls_iota == tq[None, :], logits, 0.0),
                  axis=0, keepdims=True)
    nll_sum = jnp.sum(lse - sel)

    mn = jnp.min(dist, axis=0, keepdims=True)
    pred = jnp.min(jnp.where(dist == mn, cls_iota, C), axis=0)
    pred_ref[0, 0, :] = pred
    acc_sum = jnp.sum((pred == tq).astype(jnp.float32))

    @pl.when((b == 0) & (qi == 0))
    def _init_stats():
        loss_ref[...] = jnp.zeros_like(loss_ref)
        acc_ref[...] = jnp.zeros_like(acc_ref)

    inv = 1.0 / (B * Q)
    loss_ref[...] += jnp.full((1, 128), nll_sum * inv, jnp.float32)
    acc_ref[...] += jnp.full((1, 128), acc_sum * inv, jnp.float32)


def kernel(support_embeddings, support_targets, query_embeddings,
           query_targets):
    sup_t3 = support_targets.reshape(B, 1, N)
    qt3 = query_targets.reshape(B * NQ, 1, QT)

    grid = (B, NQ)
    dist, pred3, loss_v, acc_v = pl.pallas_call(
        _tc_body,
        grid=grid,
        in_specs=[
            pl.BlockSpec((1, 1, N), lambda b, q: (b, 0, 0)),
            pl.BlockSpec((1, QT, D), lambda b, q: (b, q, 0)),
            pl.BlockSpec((1, 1, QT), lambda b, q: (b * NQ + q, 0, 0)),
            pl.BlockSpec((1, N, D), lambda b, q: (b, 0, 0)),
        ],
        out_specs=[
            pl.BlockSpec((1, C, QT), lambda b, q: (b, 0, q)),
            pl.BlockSpec((1, 1, QT), lambda b, q: (b * NQ + q, 0, 0)),
            pl.BlockSpec((1, 128), lambda b, q: (0, 0)),
            pl.BlockSpec((1, 128), lambda b, q: (0, 0)),
        ],
        out_shape=[
            jax.ShapeDtypeStruct((B, C, Q), jnp.float32),
            jax.ShapeDtypeStruct((B * NQ, 1, QT), jnp.int32),
            jax.ShapeDtypeStruct((1, 128), jnp.float32),
            jax.ShapeDtypeStruct((1, 128), jnp.float32),
        ],
        scratch_shapes=[
            pltpu.VMEM((C, D), jnp.float32),
            pltpu.VMEM((C, 1), jnp.float32),
        ],
    )(sup_t3, query_embeddings, qt3, support_embeddings)

    predictions = pred3.reshape(B, Q)
    loss = loss_v[0, 0]
    accuracy = acc_v[0, 0]
    return (loss, predictions, accuracy, dist)
